# speedup vs baseline: 7.4575x; 7.4575x over previous
import functools

import jax
import jax.numpy as jnp
from jax import lax
from jax.experimental import pallas as pl
from jax.experimental.pallas import tpu as pltpu
from jax.experimental.pallas import tpu_sc as plsc

N = 10000
D = 128
E = 320000
NC = 2
NS = 16
NW = NC * NS
PN = 10240
EPW = 10240
ROWS_PER_TILE = PN // NS
STEPS = EPW // 128
NPAD = 16


def _sc_body(xu, xi, sF, dF, sR, dR, sS, dS, accp, degp,
             idxs, idxd, rows, ones, zrows, sem, acc, deg):
    c = lax.axis_index("c")
    s = lax.axis_index("s")
    wid = s * NC + c
    row0 = s * ROWS_PER_TILE

    zero16 = jnp.zeros((16,), jnp.float32)
    one16 = jnp.ones((16,), jnp.float32)
    for j in range(8):
        ones[pl.ds(j * 16, 16)] = one16

    def _zrow(i, carry):
        for j in range(8):
            zrows[i, pl.ds(j * 16, 16)] = zero16
        return carry

    lax.fori_loop(0, zrows.shape[0], _zrow, 0)

    for rel, (x_hbm, src_hbm, dst_hbm) in enumerate(
        ((xu, sF, dF), (xu, sR, dR), (xi, sS, dS))
    ):
        def _zacc(i, carry):
            pltpu.sync_copy(zrows, acc.at[pl.ds(row0 + i * zrows.shape[0],
                                                zrows.shape[0])])
            return carry

        lax.fori_loop(0, ROWS_PER_TILE // zrows.shape[0], _zacc, 0)

        def _zdeg(i, carry):
            pltpu.sync_copy(zrows.at[0], deg.at[pl.ds(row0 + i * 128, 128)])
            return carry

        lax.fori_loop(0, ROWS_PER_TILE // 128, _zdeg, 0)
        plsc.subcore_barrier()

        pltpu.sync_copy(src_hbm.at[wid], idxs)
        pltpu.sync_copy(dst_hbm.at[wid], idxd)

        def _step(i, carry):
            pltpu.async_copy(x_hbm.at[idxs.at[i]], rows, sem).wait()
            pltpu.sync_copy(rows, acc.at[idxd.at[i]], add=True)
            pltpu.sync_copy(ones, deg.at[idxd.at[i]], add=True)
            return carry

        lax.fori_loop(0, STEPS, _step, 0)
        plsc.subcore_barrier()

        pltpu.sync_copy(acc.at[pl.ds(row0, ROWS_PER_TILE)],
                        accp.at[rel, c, pl.ds(row0, ROWS_PER_TILE)])
        pltpu.sync_copy(deg.at[pl.ds(row0, ROWS_PER_TILE)],
                        degp.at[rel, c, pl.ds(row0, ROWS_PER_TILE)])


_sc_scatter = functools.partial(
    pl.kernel,
    out_type=(
        jax.ShapeDtypeStruct((3, NC, PN, D), jnp.float32),
        jax.ShapeDtypeStruct((3, NC, PN), jnp.float32),
    ),
    mesh=plsc.VectorSubcoreMesh(core_axis_name="c", subcore_axis_name="s"),
    scratch_types=[
        pltpu.VMEM((STEPS, 128), jnp.int32),
        pltpu.VMEM((STEPS, 128), jnp.int32),
        pltpu.VMEM((128, D), jnp.float32),
        pltpu.VMEM((128,), jnp.float32),
        pltpu.VMEM((64, D), jnp.float32),
        pltpu.SemaphoreType.DMA,
        pltpu.VMEM_SHARED((PN, D), jnp.float32),
        pltpu.VMEM_SHARED((PN,), jnp.float32),
    ],
)(_sc_body)


def _tc_body(accp, degp, wf, wr, ws, ou, oi):
    def norm(r):
        a = accp[r, 0] + accp[r, 1]
        dg = jnp.maximum(degp[r, 0] + degp[r, 1], 1.0)
        return a / dg[:, None]

    ou[...] = jnp.dot(norm(0), wf[...], preferred_element_type=jnp.float32)
    oi[...] = (jnp.dot(norm(1), wr[...], preferred_element_type=jnp.float32)
               + jnp.dot(norm(2), ws[...], preferred_element_type=jnp.float32))


_TB = 1280

_tc_combine = pl.pallas_call(
    _tc_body,
    grid=(PN // _TB,),
    in_specs=[
        pl.BlockSpec((3, NC, _TB, D), lambda i: (0, 0, i, 0)),
        pl.BlockSpec((3, NC, _TB), lambda i: (0, 0, i)),
        pl.BlockSpec((D, D), lambda i: (0, 0)),
        pl.BlockSpec((D, D), lambda i: (0, 0)),
        pl.BlockSpec((D, D), lambda i: (0, 0)),
    ],
    out_specs=[
        pl.BlockSpec((_TB, D), lambda i: (i, 0)),
        pl.BlockSpec((_TB, D), lambda i: (i, 0)),
    ],
    out_shape=[
        jax.ShapeDtypeStruct((PN, D), jnp.float32),
        jax.ShapeDtypeStruct((PN, D), jnp.float32),
    ],
)


def _prep_edges(e):
    pad = N + (jnp.arange(EPW - E // NW, dtype=jnp.int32) % NPAD)
    pad = jnp.broadcast_to(pad, (NW, EPW - E // NW))

    def one(v):
        v = v.reshape(NW, E // NW)
        v = jnp.concatenate([v, pad], axis=1)
        return v.reshape(NW, STEPS, 128)

    return one(e[0]), one(e[1])


def kernel(x_user, x_item, edge_follows, edge_rates, edge_similar,
           W_follows, W_rates, W_similar):
    zpad = jnp.zeros((PN - N, D), jnp.float32)
    xu = jnp.concatenate([x_user, zpad], axis=0)
    xi = jnp.concatenate([x_item, zpad], axis=0)
    sF, dF = _prep_edges(edge_follows)
    sR, dR = _prep_edges(edge_rates)
    sS, dS = _prep_edges(edge_similar)

    accp, degp = _sc_scatter(xu, xi, sF, dF, sR, dR, sS, dS)
    ou, oi = _tc_combine(accp, degp, W_follows, W_rates, W_similar)
    return ou[:N], oi[:N]


# speedup vs baseline: 9.2668x; 1.2426x over previous
import functools

import jax
import jax.numpy as jnp
from jax import lax
from jax.experimental import pallas as pl
from jax.experimental.pallas import tpu as pltpu
from jax.experimental.pallas import tpu_sc as plsc

N = 10000
D = 128
E = 320000
NC = 2
NS = 16
NW = NC * NS
PN = 10240
EPW = 10240
ROWS_PER_TILE = PN // NS
STEPS = EPW // 128
HALF = STEPS // 2
NPAD = 16


def _sc_body(xu, xi, sF, dF, sR, dR, sS, dS, accp, degp,
             idxs, idxd, rowsA, rowsB, ones, zrows, gsemA, gsemB, dsem,
             acc, deg):
    c = lax.axis_index("c")
    s = lax.axis_index("s")
    wid = s * NC + c
    row0 = s * ROWS_PER_TILE

    zero16 = jnp.zeros((16,), jnp.float32)
    one16 = jnp.ones((16,), jnp.float32)
    for j in range(8):
        ones[pl.ds(j * 16, 16)] = one16

    def _zrow(i, carry):
        for j in range(8):
            zrows[i, pl.ds(j * 16, 16)] = zero16
        return carry

    lax.fori_loop(0, zrows.shape[0], _zrow, 0)

    for rel, (x_hbm, src_hbm, dst_hbm) in enumerate(
        ((xu, sF, dF), (xu, sR, dR), (xi, sS, dS))
    ):
        def _zacc(i, carry):
            pltpu.sync_copy(zrows, acc.at[pl.ds(row0 + i * zrows.shape[0],
                                                zrows.shape[0])])
            return carry

        lax.fori_loop(0, ROWS_PER_TILE // zrows.shape[0], _zacc, 0)

        def _zdeg(i, carry):
            pltpu.sync_copy(zrows.at[0], deg.at[pl.ds(row0 + i * 128, 128)])
            return carry

        lax.fori_loop(0, ROWS_PER_TILE // 128, _zdeg, 0)
        plsc.subcore_barrier()

        for h in range(2):
            pltpu.sync_copy(src_hbm.at[wid, pl.ds(h * HALF, HALF)], idxs)
            pltpu.sync_copy(dst_hbm.at[wid, pl.ds(h * HALF, HALF)], idxd)
            pltpu.async_copy(x_hbm.at[idxs.at[0]], rowsA, gsemA)

            def _pair(k, carry):
                i0 = 2 * k
                pltpu.make_async_copy(x_hbm.at[idxs.at[i0]], rowsA,
                                      gsemA).wait()
                pltpu.async_copy(x_hbm.at[idxs.at[i0 + 1]], rowsB, gsemB)
                pltpu.async_copy(ones, deg.at[idxd.at[i0]], dsem, add=True)
                pltpu.sync_copy(rowsA, acc.at[idxd.at[i0]], add=True)

                pltpu.make_async_copy(x_hbm.at[idxs.at[i0 + 1]], rowsB,
                                      gsemB).wait()

                @pl.when(i0 + 2 < HALF)
                def _():
                    pltpu.async_copy(x_hbm.at[idxs.at[i0 + 2]], rowsA, gsemA)

                pltpu.async_copy(ones, deg.at[idxd.at[i0 + 1]], dsem, add=True)
                pltpu.sync_copy(rowsB, acc.at[idxd.at[i0 + 1]], add=True)
                return carry

            lax.fori_loop(0, HALF // 2, _pair, 0)

            def _ddrain(i, carry):
                pltpu.make_async_copy(ones, deg.at[idxd.at[0]], dsem).wait()
                return carry

            lax.fori_loop(0, HALF, _ddrain, 0)
        plsc.subcore_barrier()

        pltpu.sync_copy(acc.at[pl.ds(row0, ROWS_PER_TILE)],
                        accp.at[rel, c, pl.ds(row0, ROWS_PER_TILE)])
        pltpu.sync_copy(deg.at[pl.ds(row0, ROWS_PER_TILE)],
                        degp.at[rel, c, pl.ds(row0, ROWS_PER_TILE)])


_sc_scatter = functools.partial(
    pl.kernel,
    out_type=(
        jax.ShapeDtypeStruct((3, NC, PN, D), jnp.float32),
        jax.ShapeDtypeStruct((3, NC, PN), jnp.float32),
    ),
    mesh=plsc.VectorSubcoreMesh(core_axis_name="c", subcore_axis_name="s"),
    scratch_types=[
        pltpu.VMEM((HALF, 128), jnp.int32),
        pltpu.VMEM((HALF, 128), jnp.int32),
        pltpu.VMEM((128, D), jnp.float32),
        pltpu.VMEM((128, D), jnp.float32),
        pltpu.VMEM((128,), jnp.float32),
        pltpu.VMEM((16, D), jnp.float32),
        pltpu.SemaphoreType.DMA,
        pltpu.SemaphoreType.DMA,
        pltpu.SemaphoreType.DMA,
        pltpu.VMEM_SHARED((PN, D), jnp.float32),
        pltpu.VMEM_SHARED((PN,), jnp.float32),
    ],
)(_sc_body)


def _tc_body(accp, degp, wf, wr, ws, ou, oi):
    def norm(r):
        a = accp[r, 0] + accp[r, 1]
        dg = jnp.maximum(degp[r, 0] + degp[r, 1], 1.0)
        return a / dg[:, None]

    ou[...] = jnp.dot(norm(0), wf[...], preferred_element_type=jnp.float32)
    oi[...] = (jnp.dot(norm(1), wr[...], preferred_element_type=jnp.float32)
               + jnp.dot(norm(2), ws[...], preferred_element_type=jnp.float32))


_TB = 1280

_tc_combine = pl.pallas_call(
    _tc_body,
    grid=(PN // _TB,),
    in_specs=[
        pl.BlockSpec((3, NC, _TB, D), lambda i: (0, 0, i, 0)),
        pl.BlockSpec((3, NC, _TB), lambda i: (0, 0, i)),
        pl.BlockSpec((D, D), lambda i: (0, 0)),
        pl.BlockSpec((D, D), lambda i: (0, 0)),
        pl.BlockSpec((D, D), lambda i: (0, 0)),
    ],
    out_specs=[
        pl.BlockSpec((_TB, D), lambda i: (i, 0)),
        pl.BlockSpec((_TB, D), lambda i: (i, 0)),
    ],
    out_shape=[
        jax.ShapeDtypeStruct((PN, D), jnp.float32),
        jax.ShapeDtypeStruct((PN, D), jnp.float32),
    ],
)


def _prep_edges(e):
    pad = N + (jnp.arange(EPW - E // NW, dtype=jnp.int32) % NPAD)
    pad = jnp.broadcast_to(pad, (NW, EPW - E // NW))

    def one(v):
        v = v.reshape(NW, E // NW)
        v = jnp.concatenate([v, pad], axis=1)
        return v.reshape(NW, STEPS, 128)

    return one(e[0]), one(e[1])


def kernel(x_user, x_item, edge_follows, edge_rates, edge_similar,
           W_follows, W_rates, W_similar):
    zpad = jnp.zeros((PN - N, D), jnp.float32)
    xu = jnp.concatenate([x_user, zpad], axis=0)
    xi = jnp.concatenate([x_item, zpad], axis=0)
    sF, dF = _prep_edges(edge_follows)
    sR, dR = _prep_edges(edge_rates)
    sS, dS = _prep_edges(edge_similar)

    accp, degp = _sc_scatter(xu, xi, sF, dF, sR, dR, sS, dS)
    ou, oi = _tc_combine(accp, degp, W_follows, W_rates, W_similar)
    return ou[:N], oi[:N]


# speedup vs baseline: 9.3261x; 1.0064x over previous
import functools

import jax
import jax.numpy as jnp
from jax import lax
from jax.experimental import pallas as pl
from jax.experimental.pallas import tpu as pltpu
from jax.experimental.pallas import tpu_sc as plsc

N = 10000
D = 128
E = 320000
NC = 2
NS = 16
NW = NC * NS
PN = 10240
EPW = 10240
ROWS_PER_TILE = PN // NS
CH = 64
STEPS = EPW // CH
QTR = STEPS // 4
NPAD = 16


def _sc_body(xu, xi, sF, dF, sR, dR, sS, dS, accp, degp,
             idxs, idxd, rows0, rows1, rows2, rows3, ones, zrows,
             gsem0, gsem1, gsem2, gsem3, ssem0, ssem1, ssem2, ssem3, dsem,
             acc, deg):
    rows = (rows0, rows1, rows2, rows3)
    gsems = (gsem0, gsem1, gsem2, gsem3)
    ssems = (ssem0, ssem1, ssem2, ssem3)
    c = lax.axis_index("c")
    s = lax.axis_index("s")
    wid = s * NC + c
    row0 = s * ROWS_PER_TILE

    zero16 = jnp.zeros((16,), jnp.float32)
    one16 = jnp.ones((16,), jnp.float32)
    for j in range(CH // 16):
        ones[pl.ds(j * 16, 16)] = one16

    def _zrow(i, carry):
        for j in range(8):
            zrows[i, pl.ds(j * 16, 16)] = zero16
        return carry

    lax.fori_loop(0, zrows.shape[0], _zrow, 0)

    ZR = zrows.shape[0]

    for rel, (x_hbm, src_hbm, dst_hbm) in enumerate(
        ((xu, sF, dF), (xu, sR, dR), (xi, sS, dS))
    ):
        def _zacc(i, carry):
            pltpu.async_copy(zrows, acc.at[pl.ds(row0 + i * ZR, ZR)], dsem)
            return carry

        lax.fori_loop(0, ROWS_PER_TILE // ZR, _zacc, 0)

        def _zdeg(i, carry):
            pltpu.async_copy(zrows.at[0], deg.at[pl.ds(row0 + i * 128, 128)],
                             dsem)
            return carry

        lax.fori_loop(0, ROWS_PER_TILE // 128, _zdeg, 0)

        def _zdrain_a(i, carry):
            pltpu.make_async_copy(zrows, acc.at[pl.ds(row0, ZR)], dsem).wait()
            return carry

        lax.fori_loop(0, ROWS_PER_TILE // ZR, _zdrain_a, 0)

        def _zdrain_d(i, carry):
            pltpu.make_async_copy(zrows.at[0], deg.at[pl.ds(row0, 128)],
                                  dsem).wait()
            return carry

        lax.fori_loop(0, ROWS_PER_TILE // 128, _zdrain_d, 0)
        plsc.subcore_barrier()

        def _gather(i, b):
            pltpu.async_copy(x_hbm.at[idxs.at[i]], rows[b], gsems[b])

        def _gwait(i, b):
            pltpu.make_async_copy(x_hbm.at[idxs.at[i]], rows[b],
                                  gsems[b]).wait()

        def _swait(b):
            pltpu.make_async_copy(rows[b], acc.at[idxd.at[0]],
                                  ssems[b]).wait()

        def _process(i, b):
            _gwait(i, b)
            pltpu.async_copy(ones, deg.at[idxd.at[i]], dsem, add=True)
            pltpu.async_copy(rows[b], acc.at[idxd.at[i]], ssems[b], add=True)

        for h in range(4):
            pltpu.sync_copy(src_hbm.at[wid, pl.ds(h * QTR, QTR)], idxs)
            pltpu.sync_copy(dst_hbm.at[wid, pl.ds(h * QTR, QTR)], idxd)

            _gather(0, 0)
            _gather(1, 1)
            _process(0, 0)
            _gather(2, 2)
            _process(1, 1)
            _gather(3, 3)

            def _quad(k, carry):
                i0 = 2 + 4 * k
                for j in range(4):
                    i = i0 + j
                    b = (2 + j) % 4
                    _process(i, b)
                    bn = (b + 2) % 4
                    _swait(bn)
                    _gather(i + 2, bn)
                return carry

            lax.fori_loop(0, (QTR - 4) // 4, _quad, 0)
            _process(QTR - 2, (QTR - 2) % 4)
            _process(QTR - 1, (QTR - 1) % 4)
            for b in range(4):
                _swait(b)

            def _ddrain(i, carry):
                pltpu.make_async_copy(ones, deg.at[idxd.at[0]], dsem).wait()
                return carry

            lax.fori_loop(0, QTR, _ddrain, 0)
        plsc.subcore_barrier()

        pltpu.sync_copy(acc.at[pl.ds(row0, ROWS_PER_TILE)],
                        accp.at[rel, c, pl.ds(row0, ROWS_PER_TILE)])
        pltpu.sync_copy(deg.at[pl.ds(row0, ROWS_PER_TILE)],
                        degp.at[rel, c, pl.ds(row0, ROWS_PER_TILE)])


_sc_scatter = functools.partial(
    pl.kernel,
    out_type=(
        jax.ShapeDtypeStruct((3, NC, PN, D), jnp.float32),
        jax.ShapeDtypeStruct((3, NC, PN), jnp.float32),
    ),
    mesh=plsc.VectorSubcoreMesh(core_axis_name="c", subcore_axis_name="s"),
    scratch_types=[
        pltpu.VMEM((QTR, CH), jnp.int32),
        pltpu.VMEM((QTR, CH), jnp.int32),
        pltpu.VMEM((CH, D), jnp.float32),
        pltpu.VMEM((CH, D), jnp.float32),
        pltpu.VMEM((CH, D), jnp.float32),
        pltpu.VMEM((CH, D), jnp.float32),
        pltpu.VMEM((CH,), jnp.float32),
        pltpu.VMEM((4, D), jnp.float32),
        pltpu.SemaphoreType.DMA,
        pltpu.SemaphoreType.DMA,
        pltpu.SemaphoreType.DMA,
        pltpu.SemaphoreType.DMA,
        pltpu.SemaphoreType.DMA,
        pltpu.SemaphoreType.DMA,
        pltpu.SemaphoreType.DMA,
        pltpu.SemaphoreType.DMA,
        pltpu.SemaphoreType.DMA,
        pltpu.VMEM_SHARED((PN, D), jnp.float32),
        pltpu.VMEM_SHARED((PN,), jnp.float32),
    ],
)(_sc_body)


def _tc_body(accp, degp, wf, wr, ws, ou, oi):
    def norm(r):
        a = accp[r, 0] + accp[r, 1]
        dg = jnp.maximum(degp[r, 0] + degp[r, 1], 1.0)
        return a / dg[:, None]

    ou[...] = jnp.dot(norm(0), wf[...], preferred_element_type=jnp.float32)
    oi[...] = (jnp.dot(norm(1), wr[...], preferred_element_type=jnp.float32)
               + jnp.dot(norm(2), ws[...], preferred_element_type=jnp.float32))


_TB = 1280

_tc_combine = pl.pallas_call(
    _tc_body,
    grid=(PN // _TB,),
    in_specs=[
        pl.BlockSpec((3, NC, _TB, D), lambda i: (0, 0, i, 0)),
        pl.BlockSpec((3, NC, _TB), lambda i: (0, 0, i)),
        pl.BlockSpec((D, D), lambda i: (0, 0)),
        pl.BlockSpec((D, D), lambda i: (0, 0)),
        pl.BlockSpec((D, D), lambda i: (0, 0)),
    ],
    out_specs=[
        pl.BlockSpec((_TB, D), lambda i: (i, 0)),
        pl.BlockSpec((_TB, D), lambda i: (i, 0)),
    ],
    out_shape=[
        jax.ShapeDtypeStruct((PN, D), jnp.float32),
        jax.ShapeDtypeStruct((PN, D), jnp.float32),
    ],
)


def _prep_edges(e):
    pad = N + (jnp.arange(EPW - E // NW, dtype=jnp.int32) % NPAD)
    pad = jnp.broadcast_to(pad, (NW, EPW - E // NW))

    def one(v):
        v = v.reshape(NW, E // NW)
        v = jnp.concatenate([v, pad], axis=1)
        return v.reshape(NW, STEPS, CH)

    return one(e[0]), one(e[1])


def kernel(x_user, x_item, edge_follows, edge_rates, edge_similar,
           W_follows, W_rates, W_similar):
    zpad = jnp.zeros((PN - N, D), jnp.float32)
    xu = jnp.concatenate([x_user, zpad], axis=0)
    xi = jnp.concatenate([x_item, zpad], axis=0)
    sF, dF = _prep_edges(edge_follows)
    sR, dR = _prep_edges(edge_rates)
    sS, dS = _prep_edges(edge_similar)

    accp, degp = _sc_scatter(xu, xi, sF, dF, sR, dR, sS, dS)
    ou, oi = _tc_combine(accp, degp, W_follows, W_rates, W_similar)
    return ou[:N], oi[:N]
